# Initial kernel scaffold; baseline (speedup 1.0000x reference)
import jax
import jax.numpy as jnp
from jax.experimental import pallas as pl


def kernel(x, word_table, token_table, pos_table, ln_gamma, ln_beta):
    raise NotImplementedError("write your pallas kernel here")


# speedup vs baseline: 3.1517x; 3.1517x over previous
import functools

import jax
import jax.numpy as jnp
from jax import lax
from jax.experimental import pallas as pl
from jax.experimental.pallas import tpu as pltpu
from jax.experimental.pallas import tpu_sc as plsc

B, S, H, V = 32, 512, 768, 21128
EPS = 1e-5
TOK = B * S
NW = 32
TPW = TOK // NW
CHUNK = 128
NCH = TPW // CHUNK


def _gather_sc(word_table, idx_flat):
    mesh = plsc.VectorSubcoreMesh(core_axis_name="c", subcore_axis_name="s")

    @functools.partial(
        pl.kernel,
        mesh=mesh,
        out_type=jax.ShapeDtypeStruct((TOK, H), jnp.float32),
        scratch_types=[
            pltpu.VMEM((CHUNK,), jnp.int32),
            pltpu.VMEM((CHUNK, H), jnp.float32),
            pltpu.SemaphoreType.DMA,
        ],
    )
    def k(table_hbm, idx_hbm, out_hbm, idx_v, rows_v, sem):
        wid = lax.axis_index("s") * 2 + lax.axis_index("c")
        base = wid * TPW

        def body(c, carry):
            off = base + c * CHUNK
            pltpu.sync_copy(idx_hbm.at[pl.ds(off, CHUNK)], idx_v)
            pltpu.async_copy(table_hbm.at[idx_v], rows_v, sem).wait()
            pltpu.sync_copy(rows_v, out_hbm.at[pl.ds(off, CHUNK)])
            return carry

        lax.fori_loop(0, NCH, body, 0)

    return k(word_table, idx_flat)


def _ln_tc(gathered, pos_table, token_row, gamma2d, beta2d):
    def body(g_ref, pos_ref, tok_ref, gam_ref, bet_ref, out_ref):
        e = g_ref[...] + pos_ref[...] + tok_ref[...]
        mu = jnp.mean(e, axis=-1, keepdims=True)
        d = e - mu
        var = jnp.mean(d * d, axis=-1, keepdims=True)
        out_ref[...] = d * lax.rsqrt(var + EPS) * gam_ref[...] + bet_ref[...]

    return pl.pallas_call(
        body,
        grid=(B,),
        in_specs=[
            pl.BlockSpec((S, H), lambda i: (i, 0)),
            pl.BlockSpec((S, H), lambda i: (0, 0)),
            pl.BlockSpec((1, H), lambda i: (0, 0)),
            pl.BlockSpec((1, H), lambda i: (0, 0)),
            pl.BlockSpec((1, H), lambda i: (0, 0)),
        ],
        out_specs=pl.BlockSpec((S, H), lambda i: (i, 0)),
        out_shape=jax.ShapeDtypeStruct((TOK, H), jnp.float32),
    )(gathered, pos_table, token_row, gamma2d, beta2d)


def kernel(x, word_table, token_table, pos_table, ln_gamma, ln_beta):
    idx_flat = x.reshape(TOK)
    gathered = _gather_sc(word_table, idx_flat)
    out = _ln_tc(
        gathered,
        pos_table[:S],
        token_table[0:1],
        ln_gamma.reshape(1, H),
        ln_beta.reshape(1, H),
    )
    return out.reshape(B, S, H)
